# baseline (speedup 1.0000x reference)
import jax
import jax.numpy as jnp
from jax import lax
from jax.experimental import pallas as pl
from jax.experimental.pallas import tpu as pltpu
from jax.experimental.pallas import tpu_sc as plsc

NC, NS, L = 2, 16, 16
NW = NC * NS


def _make_upsample(M, C):
  rows_per_w = M // NW
  CHUNK = 128
  NBUF = 7
  LA = 3
  n_chunks = rows_per_w // CHUNK
  assert n_chunks >= 2 * NBUF
  mesh = plsc.VectorSubcoreMesh(
      core_axis_name="c", subcore_axis_name="s",
      num_cores=NC, num_subcores=NS)

  def body(data_hbm, cidx_hbm, out_hbm, idx_v,
           buf0, buf1, buf2, buf3, buf4, buf5, buf6,
           gsem0, gsem1, gsem2, gsem3, gsem4, gsem5, gsem6,
           osem0, osem1, osem2, osem3, osem4, osem5, osem6):
    wid = lax.axis_index("s") * NC + lax.axis_index("c")
    base = wid * rows_per_w
    bufs = (buf0, buf1, buf2, buf3, buf4, buf5, buf6)
    gsems = (gsem0, gsem1, gsem2, gsem3, gsem4, gsem5, gsem6)
    osems = (osem0, osem1, osem2, osem3, osem4, osem5, osem6)

    pltpu.sync_copy(cidx_hbm.at[pl.ds(base, rows_per_w)], idx_v)

    def shift(g):
      for j in range(CHUNK // L):
        o = g * CHUNK + j * L
        idx_v[pl.ds(o, L)] = idx_v[pl.ds(o, L)] >> 3

    def gather(g, b):
      return pltpu.make_async_copy(
          data_hbm.at[idx_v.at[pl.ds(g * CHUNK, CHUNK)]], bufs[b], gsems[b])

    def put(g, b):
      return pltpu.make_async_copy(
          bufs[b], out_hbm.at[pl.ds(base + g * CHUNK, CHUNK)], osems[b])

    def step(g, b, wait_put, start_la):
      if wait_put:
        put(g - (NBUF - LA), (b - (NBUF - LA)) % NBUF).wait()
      if start_la:
        shift(g + LA)
        gather(g + LA, (b + LA) % NBUF).start()
      gather(g, b).wait()
      put(g, b).start()

    for b in range(LA):
      shift(b)
      gather(b, b).start()
    n_steady = (n_chunks // NBUF) - 1
    for g in range(NBUF):
      step(g, g, g >= NBUF - LA, g + LA < n_chunks)

    def ring_body(t, carry):
      for b in range(NBUF):
        step(NBUF * t + b, b, True, True)
      return carry
    lax.fori_loop(1, n_steady, ring_body, 0)

    for g in range(n_steady * NBUF, n_chunks):
      step(g, g % NBUF, g >= NBUF - LA, g + LA < n_chunks)
    for g in range(n_chunks - (NBUF - LA), n_chunks):
      put(g, g % NBUF).wait()

  return pl.kernel(
      body,
      out_type=jax.ShapeDtypeStruct((M, C), jnp.float32),
      mesh=mesh,
      scratch_types=(
          [pltpu.VMEM((rows_per_w,), jnp.int32)]
          + [pltpu.VMEM((CHUNK, C), jnp.float32)] * 7
          + [pltpu.SemaphoreType.DMA] * 14
      ),
  )


def kernel(data, child_idx, depth):
  del depth
  M, = child_idx.shape
  _, C = data.shape
  return _make_upsample(M, C)(data, child_idx)


# speedup vs baseline: 1.0062x; 1.0062x over previous
import jax
import jax.numpy as jnp
from jax import lax
from jax.experimental import pallas as pl
from jax.experimental.pallas import tpu as pltpu
from jax.experimental.pallas import tpu_sc as plsc

NC, NS, L = 2, 16, 16
NW = NC * NS


def _make_upsample(M, C):
  rows_per_w = M // NW
  CHUNK = 128
  NBUF = 6
  LA = 3
  n_chunks = rows_per_w // CHUNK
  assert n_chunks >= 2 * NBUF
  mesh = plsc.VectorSubcoreMesh(
      core_axis_name="c", subcore_axis_name="s",
      num_cores=NC, num_subcores=NS)

  def body(data_hbm, cidx_hbm, out_hbm, idx_v, pidx_v,
           buf0, buf1, buf2, buf3, buf4, buf5,
           gsem0, gsem1, gsem2, gsem3, gsem4, gsem5,
           osem0, osem1, osem2, osem3, osem4, osem5):
    wid = lax.axis_index("s") * NC + lax.axis_index("c")
    base = wid * rows_per_w
    bufs = (buf0, buf1, buf2, buf3, buf4, buf5)
    gsems = (gsem0, gsem1, gsem2, gsem3, gsem4, gsem5)
    osems = (osem0, osem1, osem2, osem3, osem4, osem5)

    pltpu.sync_copy(cidx_hbm.at[pl.ds(base, rows_per_w)], idx_v)

    def shift(g):
      for j in range(CHUNK // L):
        o = g * CHUNK + j * L
        pidx_v[pl.ds(o, L)] = idx_v[pl.ds(o, L)] >> 3

    def gather(g, b):
      return pltpu.make_async_copy(
          data_hbm.at[pidx_v.at[pl.ds(g * CHUNK, CHUNK)]], bufs[b], gsems[b])

    def put(g, b):
      return pltpu.make_async_copy(
          bufs[b], out_hbm.at[pl.ds(base + g * CHUNK, CHUNK)], osems[b])

    def step(g, b, wait_put, start_la):
      if wait_put:
        put(g - (NBUF - LA), (b - (NBUF - LA)) % NBUF).wait()
      if start_la:
        shift(g + LA)
        gather(g + LA, (b + LA) % NBUF).start()
      gather(g, b).wait()
      put(g, b).start()

    for b in range(LA):
      shift(b)
      gather(b, b).start()
    n_steady = (n_chunks // NBUF) - 1
    for g in range(NBUF):
      step(g, g, g >= NBUF - LA, g + LA < n_chunks)

    def ring_body(t, carry):
      for b in range(NBUF):
        step(NBUF * t + b, b, True, True)
      return carry
    lax.fori_loop(1, n_steady, ring_body, 0)

    for g in range(n_steady * NBUF, n_chunks):
      step(g, g % NBUF, g >= NBUF - LA, g + LA < n_chunks)
    for g in range(n_chunks - (NBUF - LA), n_chunks):
      put(g, g % NBUF).wait()

  return pl.kernel(
      body,
      out_type=jax.ShapeDtypeStruct((M, C), jnp.float32),
      mesh=mesh,
      scratch_types=(
          [pltpu.VMEM((rows_per_w,), jnp.int32),
           pltpu.VMEM((rows_per_w,), jnp.int32)]
          + [pltpu.VMEM((CHUNK, C), jnp.float32)] * 6
          + [pltpu.SemaphoreType.DMA] * 12
      ),
  )


def kernel(data, child_idx, depth):
  del depth
  M, = child_idx.shape
  _, C = data.shape
  return _make_upsample(M, C)(data, child_idx)
